# baseline (speedup 1.0000x reference)
import functools

import jax
import jax.numpy as jnp
from jax import lax
from jax.experimental import pallas as pl
from jax.experimental.pallas import tpu as pltpu
from jax.experimental.pallas import tpu_sc as plsc

INPUT_SIZE = 1024
OUTPUT_SIZE = 1024
NUM_CELLS = 16
TOP_K = 2
BATCH = 32
FEAT = INPUT_SIZE + OUTPUT_SIZE
HALF = OUTPUT_SIZE // 2
_NEG_INF = float("-inf")


def _sc_gate_body(feats_hbm, wt_hbm, bt_hbm, out_hbm, wrow_v, frow_v, bt_v,
                  gate_v):
    ncores = 2
    wid = lax.axis_index("s") * ncores + lax.axis_index("c")

    pltpu.sync_copy(wt_hbm, wrow_v)
    pltpu.sync_copy(feats_hbm.at[wid], frow_v)
    pltpu.sync_copy(bt_hbm, bt_v)

    def mac(k, acc):
        fv = frow_v[pl.ds(16 * k, 16)]
        for u in range(16):
            acc = acc + fv[u] * wrow_v[pl.ds((16 * k + u) * NUM_CELLS, 16)]
        return acc

    logits = lax.fori_loop(0, FEAT // 16, mac,
                           jnp.zeros((NUM_CELLS,), jnp.float32))
    logits = logits + bt_v[...]
    lanes = lax.iota(jnp.int32, NUM_CELLS)

    def vmax(v):
        m = jnp.zeros_like(v) + v[0]
        for u in range(1, NUM_CELLS):
            m = jnp.maximum(m, v[u])
        return m

    def first_argmax_onehot(v):
        m = vmax(v)
        iv = jnp.where(v == m, lanes, NUM_CELLS)
        im = jnp.zeros_like(iv) + iv[0]
        for u in range(1, NUM_CELLS):
            im = jnp.minimum(im, iv[u])
        return lanes == im, m

    oh1, m1 = first_argmax_onehot(logits)
    masked = jnp.where(oh1, _NEG_INF, logits)
    oh2, m2 = first_argmax_onehot(masked)
    d = jnp.exp(m2 - m1)
    p1 = 1.0 / (1.0 + d)
    p2 = d / (1.0 + d)
    gate_v[...] = jnp.where(oh1, p1, jnp.where(oh2, p2, 0.0))
    pltpu.sync_copy(gate_v, out_hbm.at[wid])


def _sc_gate(feats, W_topk, b_topk):
    mesh = plsc.VectorSubcoreMesh(core_axis_name="c", subcore_axis_name="s")
    return pl.kernel(
        _sc_gate_body,
        mesh=mesh,
        out_type=jax.ShapeDtypeStruct((BATCH, NUM_CELLS), jnp.float32),
        scratch_types=[
            pltpu.VMEM((FEAT * NUM_CELLS,), jnp.float32),
            pltpu.VMEM((FEAT,), jnp.float32),
            pltpu.VMEM((NUM_CELLS,), jnp.float32),
            pltpu.VMEM((NUM_CELLS,), jnp.float32),
        ],
    )(feats, W_topk.reshape(-1), b_topk)


def _tc_body(feats_ref, wi_ref, wj_ref, wf_ref, wo_ref, bg_ref, gate_ref,
             c_ref, outc_ref, outh_ref):
    half = pl.program_id(0)
    e = pl.program_id(1)

    @pl.when(e == 0)
    def _init():
        outc_ref[:, :] = jnp.zeros_like(outc_ref)
        outh_ref[:, :] = jnp.zeros_like(outh_ref)

    feats = feats_ref[:, :]
    gi = jnp.dot(feats, wi_ref[:, :], preferred_element_type=jnp.float32)
    gj = jnp.dot(feats, wj_ref[:, :], preferred_element_type=jnp.float32)
    gf = jnp.dot(feats, wf_ref[:, :], preferred_element_type=jnp.float32)
    go = jnp.dot(feats, wo_ref[:, :], preferred_element_type=jnp.float32)

    def bias(g):
        return jnp.where(half == 0,
                         bg_ref[0, 2 * g:2 * g + 1, :],
                         bg_ref[0, 2 * g + 1:2 * g + 2, :])
    gi = gi + bias(0)
    gj = gj + bias(1)
    gf = gf + bias(2)
    go = go + bias(3)

    new_c = jax.nn.sigmoid(gf) * c_ref[:, :] + jax.nn.sigmoid(gi) * jnp.tanh(gj)
    new_h = jax.nn.sigmoid(go) * jnp.tanh(new_c)

    rows = jax.lax.broadcasted_iota(jnp.int32, (NUM_CELLS, 1), 0)
    onehot = (rows == e).astype(jnp.float32)
    g = jnp.dot(gate_ref[:, :], onehot,
                preferred_element_type=jnp.float32)
    outc_ref[:, :] += g * new_c
    outh_ref[:, :] += g * new_h


@jax.jit
def kernel(x, c, h, W_gates, b_gates, W_topk, b_topk):
    feats = jnp.concatenate([x, h], axis=-1)
    gate = _sc_gate(feats, W_topk, b_topk)
    bg = b_gates.reshape(NUM_CELLS, 8, HALF)

    wspec = lambda g: pl.BlockSpec(
        (FEAT, HALF), lambda m, e, g=g: (0, 2 * (4 * e + g) + m))

    out_c, out_h = pl.pallas_call(
        _tc_body,
        grid=(2, NUM_CELLS),
        in_specs=[
            pl.BlockSpec((BATCH, FEAT), lambda m, e: (0, 0)),
            wspec(0), wspec(1), wspec(2), wspec(3),
            pl.BlockSpec((1, 8, HALF), lambda m, e: (e, 0, 0)),
            pl.BlockSpec((BATCH, NUM_CELLS), lambda m, e: (0, 0)),
            pl.BlockSpec((BATCH, HALF), lambda m, e: (0, m)),
        ],
        out_specs=[
            pl.BlockSpec((BATCH, HALF), lambda m, e: (0, m)),
            pl.BlockSpec((BATCH, HALF), lambda m, e: (0, m)),
        ],
        out_shape=[
            jax.ShapeDtypeStruct((BATCH, OUTPUT_SIZE), jnp.float32),
            jax.ShapeDtypeStruct((BATCH, OUTPUT_SIZE), jnp.float32),
        ],
        compiler_params=pltpu.CompilerParams(
            dimension_semantics=("arbitrary", "arbitrary"),
            vmem_limit_bytes=60 * 1024 * 1024,
        ),
    )(feats, W_gates, W_gates, W_gates, W_gates, bg, gate, c)

    return (out_h, (out_c, out_h))


# speedup vs baseline: 1.0073x; 1.0073x over previous
import functools

import jax
import jax.numpy as jnp
from jax import lax
from jax.experimental import pallas as pl
from jax.experimental.pallas import tpu as pltpu
from jax.experimental.pallas import tpu_sc as plsc

INPUT_SIZE = 1024
OUTPUT_SIZE = 1024
NUM_CELLS = 16
TOP_K = 2
BATCH = 32
FEAT = INPUT_SIZE + OUTPUT_SIZE
HALF = OUTPUT_SIZE // 2
_NEG_INF = float("-inf")


def _sc_gate_body(feats_hbm, wt_hbm, bt_hbm, out_hbm, wrow_v, frow_v, bt_v,
                  gate_v):
    ncores = 2
    wid = lax.axis_index("s") * ncores + lax.axis_index("c")

    pltpu.sync_copy(wt_hbm, wrow_v)
    pltpu.sync_copy(feats_hbm.at[wid], frow_v)
    pltpu.sync_copy(bt_hbm, bt_v)

    def mac(k, acc):
        fv = frow_v[pl.ds(16 * k, 16)]
        for u in range(16):
            acc = acc + fv[u] * wrow_v[pl.ds((16 * k + u) * NUM_CELLS, 16)]
        return acc

    logits = lax.fori_loop(0, FEAT // 16, mac,
                           jnp.zeros((NUM_CELLS,), jnp.float32))
    logits = logits + bt_v[...]
    lanes = lax.iota(jnp.int32, NUM_CELLS)

    def vmax(v):
        m = jnp.zeros_like(v) + v[0]
        for u in range(1, NUM_CELLS):
            m = jnp.maximum(m, v[u])
        return m

    def first_argmax_onehot(v):
        m = vmax(v)
        iv = jnp.where(v == m, lanes, NUM_CELLS)
        im = jnp.zeros_like(iv) + iv[0]
        for u in range(1, NUM_CELLS):
            im = jnp.minimum(im, iv[u])
        return lanes == im, m

    oh1, m1 = first_argmax_onehot(logits)
    masked = jnp.where(oh1, _NEG_INF, logits)
    oh2, m2 = first_argmax_onehot(masked)
    d = jnp.exp(m2 - m1)
    p1 = 1.0 / (1.0 + d)
    p2 = d / (1.0 + d)
    gate_v[...] = jnp.where(oh1, p1, jnp.where(oh2, p2, 0.0))
    pltpu.sync_copy(gate_v, out_hbm.at[wid])


def _sc_gate(feats, W_topk, b_topk):
    mesh = plsc.VectorSubcoreMesh(core_axis_name="c", subcore_axis_name="s")
    return pl.kernel(
        _sc_gate_body,
        mesh=mesh,
        out_type=jax.ShapeDtypeStruct((BATCH, NUM_CELLS), jnp.float32),
        scratch_types=[
            pltpu.VMEM((FEAT * NUM_CELLS,), jnp.float32),
            pltpu.VMEM((FEAT,), jnp.float32),
            pltpu.VMEM((NUM_CELLS,), jnp.float32),
            pltpu.VMEM((NUM_CELLS,), jnp.float32),
        ],
    )(feats, W_topk.reshape(-1), b_topk)


NA = 4
NB = NUM_CELLS - NA


def _lstm_block(feats_ref, wi_ref, wj_ref, wf_ref, wo_ref, bg_ref, c_ref,
                half):
    feats = feats_ref[:, :]
    gi = jnp.dot(feats, wi_ref[:, :], preferred_element_type=jnp.float32)
    gj = jnp.dot(feats, wj_ref[:, :], preferred_element_type=jnp.float32)
    gf = jnp.dot(feats, wf_ref[:, :], preferred_element_type=jnp.float32)
    go = jnp.dot(feats, wo_ref[:, :], preferred_element_type=jnp.float32)

    def bias(g):
        return jnp.where(half == 0,
                         bg_ref[0, 2 * g:2 * g + 1, :],
                         bg_ref[0, 2 * g + 1:2 * g + 2, :])
    gi = gi + bias(0)
    gj = gj + bias(1)
    gf = gf + bias(2)
    go = go + bias(3)

    new_c = jax.nn.sigmoid(gf) * c_ref[:, :] + jax.nn.sigmoid(gi) * jnp.tanh(gj)
    new_h = jax.nn.sigmoid(go) * jnp.tanh(new_c)
    return new_c, new_h


def _gate_col(gate_ref, e):
    rows = jax.lax.broadcasted_iota(jnp.int32, (NUM_CELLS, 1), 0)
    onehot = (rows == e).astype(jnp.float32)
    return jnp.dot(gate_ref[:, :], onehot,
                   preferred_element_type=jnp.float32)


def _tca_body(feats_ref, wi_ref, wj_ref, wf_ref, wo_ref, bg_ref, c_ref,
              cc_ref, ch_ref):
    half = pl.program_id(0)
    new_c, new_h = _lstm_block(feats_ref, wi_ref, wj_ref, wf_ref, wo_ref,
                               bg_ref, c_ref, half)
    cc_ref[0, :, :] = new_c
    ch_ref[0, :, :] = new_h


def _tcb_body(feats_ref, wi_ref, wj_ref, wf_ref, wo_ref, bg_ref, gate_ref,
              c_ref, cca_ref, cha_ref, outc_ref, outh_ref):
    half = pl.program_id(0)
    e = pl.program_id(1)

    @pl.when(e == 0)
    def _init():
        acc_c = jnp.zeros_like(outc_ref)
        acc_h = jnp.zeros_like(outh_ref)
        for ep in range(NA):
            g = _gate_col(gate_ref, ep)
            acc_c += g * cca_ref[ep, :, :]
            acc_h += g * cha_ref[ep, :, :]
        outc_ref[:, :] = acc_c
        outh_ref[:, :] = acc_h

    new_c, new_h = _lstm_block(feats_ref, wi_ref, wj_ref, wf_ref, wo_ref,
                               bg_ref, c_ref, half)
    g = _gate_col(gate_ref, e + NA)
    outc_ref[:, :] += g * new_c
    outh_ref[:, :] += g * new_h


@jax.jit
def kernel(x, c, h, W_gates, b_gates, W_topk, b_topk):
    feats = jnp.concatenate([x, h], axis=-1)
    gate = _sc_gate(feats, W_topk, b_topk)
    bg = b_gates.reshape(NUM_CELLS, 8, HALF)

    wspec_a = lambda g: pl.BlockSpec(
        (FEAT, HALF), lambda m, e, g=g: (0, 2 * (4 * e + g) + m))
    wspec_b = lambda g: pl.BlockSpec(
        (FEAT, HALF), lambda m, e, g=g: (0, 2 * (4 * (e + NA) + g) + m))

    cc_a, ch_a = pl.pallas_call(
        _tca_body,
        grid=(2, NA),
        in_specs=[
            pl.BlockSpec((BATCH, FEAT), lambda m, e: (0, 0)),
            wspec_a(0), wspec_a(1), wspec_a(2), wspec_a(3),
            pl.BlockSpec((1, 8, HALF), lambda m, e: (e, 0, 0)),
            pl.BlockSpec((BATCH, HALF), lambda m, e: (0, m)),
        ],
        out_specs=[
            pl.BlockSpec((1, BATCH, HALF), lambda m, e: (e, 0, m)),
            pl.BlockSpec((1, BATCH, HALF), lambda m, e: (e, 0, m)),
        ],
        out_shape=[
            jax.ShapeDtypeStruct((NA, BATCH, OUTPUT_SIZE), jnp.float32),
            jax.ShapeDtypeStruct((NA, BATCH, OUTPUT_SIZE), jnp.float32),
        ],
        compiler_params=pltpu.CompilerParams(
            dimension_semantics=("arbitrary", "arbitrary"),
            vmem_limit_bytes=60 * 1024 * 1024,
        ),
    )(feats, W_gates, W_gates, W_gates, W_gates, bg, c)

    out_c, out_h = pl.pallas_call(
        _tcb_body,
        grid=(2, NB),
        in_specs=[
            pl.BlockSpec((BATCH, FEAT), lambda m, e: (0, 0)),
            wspec_b(0), wspec_b(1), wspec_b(2), wspec_b(3),
            pl.BlockSpec((1, 8, HALF), lambda m, e: (e + NA, 0, 0)),
            pl.BlockSpec((BATCH, NUM_CELLS), lambda m, e: (0, 0)),
            pl.BlockSpec((BATCH, HALF), lambda m, e: (0, m)),
            pl.BlockSpec((NA, BATCH, HALF), lambda m, e: (0, 0, m)),
            pl.BlockSpec((NA, BATCH, HALF), lambda m, e: (0, 0, m)),
        ],
        out_specs=[
            pl.BlockSpec((BATCH, HALF), lambda m, e: (0, m)),
            pl.BlockSpec((BATCH, HALF), lambda m, e: (0, m)),
        ],
        out_shape=[
            jax.ShapeDtypeStruct((BATCH, OUTPUT_SIZE), jnp.float32),
            jax.ShapeDtypeStruct((BATCH, OUTPUT_SIZE), jnp.float32),
        ],
        compiler_params=pltpu.CompilerParams(
            dimension_semantics=("arbitrary", "arbitrary"),
            vmem_limit_bytes=60 * 1024 * 1024,
        ),
    )(feats, W_gates, W_gates, W_gates, W_gates, bg, gate, c, cc_a, ch_a)

    return (out_h, (out_c, out_h))


# speedup vs baseline: 1.1562x; 1.1478x over previous
import jax
import jax.numpy as jnp
from jax.experimental import pallas as pl
from jax.experimental.pallas import tpu as pltpu

INPUT_SIZE = 1024
OUTPUT_SIZE = 1024
NUM_CELLS = 16
TOP_K = 2
BATCH = 32
FEAT = INPUT_SIZE + OUTPUT_SIZE
HALF = OUTPUT_SIZE // 2


def _body(feats_ref, wi0_ref, wi1_ref, wj0_ref, wj1_ref, wf0_ref, wf1_ref,
          wo0_ref, wo1_ref, bg_ref, wt_ref, bt_ref,
          c_ref, outc_ref, outh_ref, gate_scr):
    half = pl.program_id(0)
    e = pl.program_id(1)

    @pl.when(e == 0)
    def _init():
        logits = jnp.dot(feats_ref[:, :], wt_ref[:, :],
                         preferred_element_type=jnp.float32)
        logits = logits + bt_ref[0, :, :]
        idx1 = jnp.argmax(logits, axis=-1)[:, None]
        cols = jax.lax.broadcasted_iota(jnp.int32, (BATCH, NUM_CELLS), 1)
        oh1 = (cols == idx1)
        m1 = jnp.max(logits, axis=-1, keepdims=True)
        masked = jnp.where(oh1, -jnp.inf, logits)
        idx2 = jnp.argmax(masked, axis=-1)[:, None]
        oh2 = (cols == idx2)
        m2 = jnp.max(masked, axis=-1, keepdims=True)
        e2 = jnp.exp(m2 - m1)
        p1 = 1.0 / (1.0 + e2)
        p2 = e2 / (1.0 + e2)
        gate_scr[:, :] = jnp.where(oh1, p1, 0.0) + jnp.where(oh2, p2, 0.0)
        outc_ref[:, :] = jnp.zeros_like(outc_ref)
        outh_ref[:, :] = jnp.zeros_like(outh_ref)

    feats = feats_ref[:, :]

    def mm(a_ref, b_ref):
        return jnp.concatenate(
            [jnp.dot(feats, a_ref[:, :], preferred_element_type=jnp.float32),
             jnp.dot(feats, b_ref[:, :], preferred_element_type=jnp.float32)],
            axis=1)

    gi = mm(wi0_ref, wi1_ref)
    gj = mm(wj0_ref, wj1_ref)
    gf = mm(wf0_ref, wf1_ref)
    go = mm(wo0_ref, wo1_ref)

    def bias(g):
        return jnp.where(half == 0,
                         bg_ref[0, 2 * g:2 * g + 1, :],
                         bg_ref[0, 2 * g + 1:2 * g + 2, :])
    gi = gi + bias(0)
    gj = gj + bias(1)
    gf = gf + bias(2)
    go = go + bias(3)

    new_c = jax.nn.sigmoid(gf) * c_ref[:, :] + jax.nn.sigmoid(gi) * jnp.tanh(gj)
    new_h = jax.nn.sigmoid(go) * jnp.tanh(new_c)

    rows = jax.lax.broadcasted_iota(jnp.int32, (NUM_CELLS, 1), 0)
    onehot = (rows == e).astype(jnp.float32)
    g = jnp.dot(gate_scr[:, :], onehot,
                preferred_element_type=jnp.float32)
    outc_ref[:, :] += g * new_c
    outh_ref[:, :] += g * new_h


@jax.jit
def kernel(x, c, h, W_gates, b_gates, W_topk, b_topk):
    feats = jnp.concatenate([x, h], axis=-1)
    bg = b_gates.reshape(NUM_CELLS, 8, HALF)
    bt = b_topk.reshape(1, 1, NUM_CELLS)

    Q = HALF // 2
    wspec = lambda g, lr: pl.BlockSpec(
        (FEAT, Q), lambda m, e, g=g, lr=lr: (0, 4 * (4 * e + g) + 2 * m + lr))

    out_c, out_h = pl.pallas_call(
        _body,
        grid=(2, NUM_CELLS),
        in_specs=[
            pl.BlockSpec((BATCH, FEAT), lambda m, e: (0, 0)),
            wspec(0, 0), wspec(0, 1), wspec(1, 0), wspec(1, 1),
            wspec(2, 0), wspec(2, 1), wspec(3, 0), wspec(3, 1),
            pl.BlockSpec((1, 8, HALF), lambda m, e: (e, 0, 0)),
            pl.BlockSpec((FEAT, NUM_CELLS), lambda m, e: (0, 0)),
            pl.BlockSpec((1, 1, NUM_CELLS), lambda m, e: (0, 0, 0)),
            pl.BlockSpec((BATCH, HALF), lambda m, e: (0, m)),
        ],
        out_specs=[
            pl.BlockSpec((BATCH, HALF), lambda m, e: (0, m)),
            pl.BlockSpec((BATCH, HALF), lambda m, e: (0, m)),
        ],
        out_shape=[
            jax.ShapeDtypeStruct((BATCH, OUTPUT_SIZE), jnp.float32),
            jax.ShapeDtypeStruct((BATCH, OUTPUT_SIZE), jnp.float32),
        ],
        scratch_shapes=[
            pltpu.VMEM((BATCH, NUM_CELLS), jnp.float32),
        ],
        compiler_params=pltpu.CompilerParams(
            dimension_semantics=("arbitrary", "arbitrary"),
            vmem_limit_bytes=60 * 1024 * 1024,
        ),
    )(feats, W_gates, W_gates, W_gates, W_gates,
      W_gates, W_gates, W_gates, W_gates, bg, W_topk, bt, c)

    return (out_h, (out_c, out_h))
